# baseline (speedup 1.0000x reference)
import functools

import jax
import jax.numpy as jnp
from jax import lax
from jax.experimental import pallas as pl
from jax.experimental.pallas import tpu as pltpu
from jax.experimental.pallas import tpu_sc as plsc

B, S, D = 1024, 200, 64
N = B * S
CW = 128
NC, NS = 2, 16
NW = NC * NS
CPW = N // (NW * CW)

@functools.cache
def _make_sc_gather():
    mesh = plsc.VectorSubcoreMesh(core_axis_name="c", subcore_axis_name="s")

    @functools.partial(
        pl.kernel,
        mesh=mesh,
        compiler_params=pltpu.CompilerParams(use_tc_tiling_on_sc=False),
        out_type=jax.ShapeDtypeStruct((N, D), jnp.float32),
        scratch_types=[
            pltpu.VMEM((CPW * CW,), jnp.int32),
            pltpu.VMEM((CW, D), jnp.float32),
            pltpu.VMEM((CW, D), jnp.float32),
            pltpu.SemaphoreType.DMA,
            pltpu.SemaphoreType.DMA,
        ],
    )
    def sc_gather(table_hbm, idx_hbm, out_hbm, idx_v, buf0, buf1, sem0, sem1):
        wid = lax.axis_index("s") * NC + lax.axis_index("c")
        base = wid * (CPW * CW)
        pltpu.sync_copy(idx_hbm.at[pl.ds(base, CPW * CW)], idx_v)

        def body(jj, carry):
            j0 = 2 * jj
            c0 = pltpu.async_copy(
                table_hbm.at[idx_v.at[pl.ds(j0 * CW, CW)]], buf0, sem0)
            c1 = pltpu.async_copy(
                table_hbm.at[idx_v.at[pl.ds((j0 + 1) * CW, CW)]], buf1, sem1)
            c0.wait()
            pltpu.sync_copy(buf0, out_hbm.at[pl.ds(base + j0 * CW, CW)])
            c1.wait()
            pltpu.sync_copy(buf1, out_hbm.at[pl.ds(base + (j0 + 1) * CW, CW)])
            return carry

        lax.fori_loop(0, CPW // 2, body, 0)

    return sc_gather


BB = 128


def _tc_body(g_ref, w_ref, b_ref, attn_ref, out_ref):
    g = g_ref[...]
    w = w_ref[...]
    bias = b_ref[0]
    y = jnp.sum(g * w[0][None, None, :], axis=-1) + bias
    y = jnp.tanh(y)
    m = jnp.max(y, axis=1, keepdims=True)
    e = jnp.exp(y - m)
    a = e / jnp.sum(e, axis=1, keepdims=True)
    attn_ref[...] = a
    out_ref[...] = jnp.sum(g * a[:, :, None], axis=1)


def _tc_fused(g3, w2, bias):
    return pl.pallas_call(
        _tc_body,
        grid=(B // BB,),
        in_specs=[
            pl.BlockSpec((BB, S, D), lambda i: (i, 0, 0)),
            pl.BlockSpec((1, D), lambda i: (0, 0)),
            pl.BlockSpec(memory_space=pltpu.SMEM),
        ],
        out_specs=[
            pl.BlockSpec((BB, S), lambda i: (i, 0)),
            pl.BlockSpec((BB, D), lambda i: (i, 0)),
        ],
        out_shape=[
            jax.ShapeDtypeStruct((B, S), jnp.float32),
            jax.ShapeDtypeStruct((B, D), jnp.float32),
        ],
    )(g3, w2, bias)


def kernel(input, table, context_weight, context_bias):
    ids = input.astype(jnp.int32).reshape(N)
    g = _make_sc_gather()(table, ids)
    g3 = g.reshape(B, S, D)
    w2 = context_weight.reshape(1, D)
    attn, out = _tc_fused(g3, w2, context_bias)
    return out[None], attn
